# Initial kernel scaffold; baseline (speedup 1.0000x reference)
import jax
import jax.numpy as jnp
from jax.experimental import pallas as pl


def kernel(seq_embeds):
    raise NotImplementedError("write your pallas kernel here")


# speedup vs baseline: 1.4769x; 1.4769x over previous
import math

import jax
import jax.numpy as jnp
from jax.experimental import pallas as pl

EMBED_DIM = 512
MAX_SEQ_LEN = 1024
HALF_DIM = EMBED_DIM // 2
SCALE = math.log(10000.0) / HALF_DIM


def _pos_table_body(out_ref):
    pos = jax.lax.broadcasted_iota(
        jnp.int32, (MAX_SEQ_LEN, EMBED_DIM), 0).astype(jnp.float32)
    col = jax.lax.broadcasted_iota(jnp.int32, (MAX_SEQ_LEN, EMBED_DIM), 1)
    k = jnp.right_shift(col, 1).astype(jnp.float32)
    inv_freq = jnp.exp(k * (-SCALE))
    ang = pos * inv_freq
    even = (col & 1) == 0
    out_ref[...] = jnp.where(even, jnp.sin(ang), jnp.cos(ang))


def kernel(seq_embeds):
    del seq_embeds
    return pl.pallas_call(
        _pos_table_body,
        out_shape=jax.ShapeDtypeStruct((MAX_SEQ_LEN, EMBED_DIM), jnp.float32),
    )()


# speedup vs baseline: 6.0309x; 4.0836x over previous
import math

import jax
import jax.numpy as jnp
from jax.experimental import pallas as pl

EMBED_DIM = 512
MAX_SEQ_LEN = 1024
HALF_DIM = EMBED_DIM // 2
SCALE = math.log(10000.0) / HALF_DIM
COARSE = 32


def _pos_table_body(out_ref):
    col = jax.lax.broadcasted_iota(jnp.int32, (COARSE, EMBED_DIM), 1)
    k = jnp.right_shift(col, 1).astype(jnp.float32)
    inv_freq = jnp.exp(k * (-SCALE))
    row = jax.lax.broadcasted_iota(
        jnp.int32, (COARSE, EMBED_DIM), 0).astype(jnp.float32)
    ang_a = (row * float(COARSE)) * inv_freq
    ang_b = row * inv_freq
    even = (col & 1) == 0
    sin_a, cos_a = jnp.sin(ang_a), jnp.cos(ang_a)
    x = jnp.where(even, sin_a, cos_a)
    y = jnp.where(even, cos_a, -sin_a)
    sin_b, cos_b = jnp.sin(ang_b), jnp.cos(ang_b)
    out3 = (x[:, None, :] * cos_b[None, :, :]
            + y[:, None, :] * sin_b[None, :, :])
    out_ref[...] = out3.reshape(MAX_SEQ_LEN, EMBED_DIM)


def kernel(seq_embeds):
    del seq_embeds
    return pl.pallas_call(
        _pos_table_body,
        out_shape=jax.ShapeDtypeStruct((MAX_SEQ_LEN, EMBED_DIM), jnp.float32),
    )()
